# Initial kernel scaffold; baseline (speedup 1.0000x reference)
import jax
import jax.numpy as jnp
from jax import lax
from jax.experimental import pallas as pl
from jax.experimental.pallas import tpu as pltpu, tpu_sc as plsc

_RADIUS2 = 0.3 * 0.3
_MAX = 16
_B = 8
_N = 2048
_ROWS = 9
_CPT = 512
_WORDS_PER_CENTER = _ROWS * _MAX
_WORDS_PER_TILE = _CPT * _WORDS_PER_CENTER
_CHUNKS = _N // 16


def _tile_body(pcs_hbm, out_hbm, pts, obuf, counts):
    info = plsc.get_sparse_core_info()
    nc = info.num_cores
    wid = lax.axis_index("s") * nc + lax.axis_index("c")
    batch = wid // 4
    base_center = (wid % 4) * _CPT

    pltpu.sync_copy(pcs_hbm.at[batch], pts)

    lanes = lax.iota(jnp.int32, 16)

    def center_body(c, _):
        i = base_center + c
        cx = pts[0, i]
        cy = pts[1, i]
        cz = pts[2, i]
        obase = c * _WORDS_PER_CENTER

        fill = jnp.full((16,), i, jnp.int32)
        for r in range(_ROWS):
            obuf[pl.ds(obase + r * 16, 16)] = fill
        for o in range(8):
            counts[o] = 0

        def chunk_body(k, carry):
            j0 = k * 16
            xv = pts[0, pl.ds(j0, 16)]
            yv = pts[1, pl.ds(j0, 16)]
            zv = pts[2, pl.ds(j0, 16)]
            dx = xv - cx
            dy = yv - cy
            dz = zv - cz
            d2 = dx * dx + dy * dy + dz * dz
            jvec = j0 + lanes
            valid = (d2 <= _RADIUS2) & (jvec != i)
            nv = jnp.sum(valid.astype(jnp.int32))

            @pl.when(nv > 0)
            def _():
                oct_id = (
                    (dx > 0).astype(jnp.int32) * 4
                    + (dy > 0).astype(jnp.int32) * 2
                    + (dz > 0).astype(jnp.int32)
                )
                for o in range(8):
                    m = valid & (oct_id == o)
                    inc = jnp.sum(m.astype(jnp.int32))

                    @pl.when(inc > 0)
                    def _(o=o, m=m, inc=inc):
                        cnt = counts[o]
                        pos = jnp.cumsum(m.astype(jnp.int32))
                        slot = cnt + pos - 1
                        keep = m & (slot < _MAX)
                        slot_c = jnp.where(keep, slot, 0)
                        addr = obase + o * 16 + slot_c
                        plsc.store_scatter(obuf, [addr], jvec, mask=keep)
                        counts[o] = cnt + inc

            return carry

        lax.fori_loop(0, _CHUNKS, chunk_body, 0)
        return _

    lax.fori_loop(0, _CPT, center_body, 0)

    pltpu.sync_copy(obuf, out_hbm.at[pl.ds(wid * _WORDS_PER_TILE, _WORDS_PER_TILE)])


@jax.jit
def kernel(pcs):
    mesh = plsc.VectorSubcoreMesh(core_axis_name="c", subcore_axis_name="s")
    flat = pl.kernel(
        _tile_body,
        out_type=jax.ShapeDtypeStruct((_B * _N * _WORDS_PER_CENTER,), jnp.int32),
        mesh=mesh,
        scratch_types=[
            pltpu.VMEM((3, _N), jnp.float32),
            pltpu.VMEM((_WORDS_PER_TILE,), jnp.int32),
            pltpu.SMEM((8,), jnp.int32),
        ],
    )(pcs)
    return flat.reshape(_B, _N, _ROWS, _MAX)


# speedup vs baseline: 430.0323x; 430.0323x over previous
import jax
import jax.numpy as jnp
from jax import lax
from jax.experimental import pallas as pl
from jax.experimental.pallas import tpu as pltpu, tpu_sc as plsc

_RADIUS2 = 0.3 * 0.3
_MAX = 16
_B = 8
_N = 2048
_ROWS = 9
_CPT = 512
_WORDS_PER_CENTER = _ROWS * _MAX
_WORDS_PER_TILE = _CPT * _WORDS_PER_CENTER
_CHUNKS = _N // 16


def _tile_body(pcs_hbm, out_hbm, ptx, pty, ptz, obuf, counts):
    info = plsc.get_sparse_core_info()
    nc = info.num_cores
    wid = lax.axis_index("s") * nc + lax.axis_index("c")
    batch = wid // 4
    base_center = (wid % 4) * _CPT

    pbase = batch * (3 * _N)
    pltpu.sync_copy(pcs_hbm.at[pl.ds(pbase, _N)], ptx)
    pltpu.sync_copy(pcs_hbm.at[pl.ds(pbase + _N, _N)], pty)
    pltpu.sync_copy(pcs_hbm.at[pl.ds(pbase + 2 * _N, _N)], ptz)

    lanes = lax.iota(jnp.int32, 16)

    def center_body(c, _):
        i = base_center + c
        iv = jnp.full((16,), i, jnp.int32)
        cx = plsc.load_gather(ptx, [iv])
        cy = plsc.load_gather(pty, [iv])
        cz = plsc.load_gather(ptz, [iv])
        obase = c * _WORDS_PER_CENTER

        fill = iv
        for r in range(_ROWS):
            obuf[pl.ds(obase + r * 16, 16)] = fill
        for o in range(8):
            counts[o] = 0

        def chunk_body(k, carry):
            j0 = k * 16
            xv = ptx[pl.ds(j0, 16)]
            yv = pty[pl.ds(j0, 16)]
            zv = ptz[pl.ds(j0, 16)]
            dx = xv - cx
            dy = yv - cy
            dz = zv - cz
            d2 = dx * dx + dy * dy + dz * dz
            jvec = j0 + lanes
            valid = (d2 <= _RADIUS2) & (jvec != i)
            nv = jnp.sum(valid.astype(jnp.int32))

            @pl.when(nv > 0)
            def _():
                oct_id = (
                    (dx > 0).astype(jnp.int32) * 4
                    + (dy > 0).astype(jnp.int32) * 2
                    + (dz > 0).astype(jnp.int32)
                )
                for o in range(8):
                    m = valid & (oct_id == o)
                    inc = jnp.sum(m.astype(jnp.int32))

                    @pl.when(inc > 0)
                    def _(o=o, m=m, inc=inc):
                        cnt = counts[o]
                        pos = jnp.cumsum(m.astype(jnp.int32))
                        slot = cnt + pos - 1
                        keep = m & (slot < _MAX)
                        slot_c = jnp.where(keep, slot, 0)
                        addr = obase + o * 16 + slot_c
                        plsc.store_scatter(obuf, [addr], jvec, mask=keep)
                        counts[o] = cnt + inc

            return carry

        lax.fori_loop(0, _CHUNKS, chunk_body, 0)
        return _

    lax.fori_loop(0, _CPT, center_body, 0)

    pltpu.sync_copy(obuf, out_hbm.at[pl.ds(wid * _WORDS_PER_TILE, _WORDS_PER_TILE)])


@jax.jit
def kernel(pcs):
    mesh = plsc.VectorSubcoreMesh(core_axis_name="c", subcore_axis_name="s")
    flat = pl.kernel(
        _tile_body,
        out_type=jax.ShapeDtypeStruct((_B * _N * _WORDS_PER_CENTER,), jnp.int32),
        mesh=mesh,
        compiler_params=pltpu.CompilerParams(needs_layout_passes=False),
        scratch_types=[
            pltpu.VMEM((_N,), jnp.float32),
            pltpu.VMEM((_N,), jnp.float32),
            pltpu.VMEM((_N,), jnp.float32),
            pltpu.VMEM((_WORDS_PER_TILE,), jnp.int32),
            pltpu.SMEM((8,), jnp.int32),
        ],
    )(pcs.reshape(-1))
    return flat.reshape(_B, _N, _ROWS, _MAX)
